# baseline (speedup 1.0000x reference)
import functools

import jax
import jax.numpy as jnp
from jax import lax
from jax.experimental import pallas as pl
from jax.experimental.pallas import tpu as pltpu
from jax.experimental.pallas import tpu_sc as plsc

N = 10000
D = 128
E0 = 320000
E = 2 * E0
NC = 2
NS = 16
NW = NC * NS
CHUNK = 128
STEPS = 160
GS = 16
EP = NW * STEPS * CHUNK
PAD = EP - E
NDUMP = 240
NP = N + NDUMP
RPT = NP // NS
RB = 400

_mesh = plsc.VectorSubcoreMesh(core_axis_name="c", subcore_axis_name="s")


def _wid():
    return lax.axis_index("c") * NS + lax.axis_index("s")


@functools.partial(
    pl.kernel,
    out_type=jax.ShapeDtypeStruct((NC, NP, D), jnp.float32),
    mesh=_mesh,
    scratch_types=[
        pltpu.VMEM((STEPS, CHUNK), jnp.int32),
        pltpu.VMEM((CHUNK, D), jnp.float32),
        pltpu.VMEM_SHARED((NP, D), jnp.float32),
    ],
    name="gcn_deg_sc",
)
def _deg_sc(dst_hbm, ones_hbm, zeros_hbm, out_hbm, didx, ones_v, acc):
    c = lax.axis_index("c")
    t = lax.axis_index("s")
    w = _wid()
    pltpu.sync_copy(dst_hbm.at[w], didx)
    pltpu.sync_copy(ones_hbm, ones_v)
    pltpu.sync_copy(zeros_hbm, acc.at[pl.ds(t * RPT, RPT)])
    plsc.subcore_barrier()

    def body(j, carry):
        pltpu.sync_copy(ones_v, acc.at[didx.at[j]], add=True)
        return carry

    lax.fori_loop(0, STEPS, body, 0, unroll=False)
    plsc.subcore_barrier()
    pltpu.sync_copy(acc.at[pl.ds(t * RPT, RPT)], out_hbm.at[c, pl.ds(t * RPT, RPT)])


@functools.partial(
    pl.kernel,
    out_type=jax.ShapeDtypeStruct((NC, NP, D), jnp.float32),
    mesh=_mesh,
    scratch_types=[
        pltpu.VMEM((GS, CHUNK), jnp.int32),
        pltpu.VMEM((GS, CHUNK), jnp.int32),
        pltpu.VMEM((CHUNK, D), jnp.float32),
        pltpu.VMEM((CHUNK, D), jnp.float32),
        pltpu.VMEM_SHARED((NP, D), jnp.float32),
        pltpu.SemaphoreType.DMA,
        pltpu.SemaphoreType.DMA,
        pltpu.SemaphoreType.DMA,
        pltpu.SemaphoreType.DMA,
    ],
    name="gcn_agg_sc",
)
def _agg_sc(u_hbm, src_hbm, dst_hbm, zeros_hbm, out_hbm,
            sidx, didx, rows0, rows1, acc, g0, g1, c0, c1):
    c = lax.axis_index("c")
    t = lax.axis_index("s")
    w = _wid()
    pltpu.sync_copy(zeros_hbm, acc.at[pl.ds(t * RPT, RPT)])
    plsc.subcore_barrier()

    def group(g, carry):
        pltpu.sync_copy(src_hbm.at[w, pl.ds(g * GS, GS)], sidx)
        pltpu.sync_copy(dst_hbm.at[w, pl.ds(g * GS, GS)], didx)
        pltpu.async_copy(u_hbm.at[sidx.at[0]], rows0, g0)
        pltpu.async_copy(u_hbm.at[sidx.at[1]], rows1, g1)

        def body(i, carry):
            j0 = 2 * i
            pltpu.make_async_copy(u_hbm.at[sidx.at[j0]], rows0, g0).wait()
            pltpu.async_copy(rows0, acc.at[didx.at[j0]], c0, add=True)
            pltpu.make_async_copy(u_hbm.at[sidx.at[j0 + 1]], rows1, g1).wait()
            pltpu.async_copy(rows1, acc.at[didx.at[j0 + 1]], c1, add=True)

            @pl.when(i < GS // 2 - 1)
            def _():
                pltpu.make_async_copy(rows0, acc.at[didx.at[j0]], c0).wait()
                pltpu.async_copy(u_hbm.at[sidx.at[j0 + 2]], rows0, g0)
                pltpu.make_async_copy(rows1, acc.at[didx.at[j0 + 1]], c1).wait()
                pltpu.async_copy(u_hbm.at[sidx.at[j0 + 3]], rows1, g1)

            return carry

        lax.fori_loop(0, GS // 2, body, 0, unroll=False)
        pltpu.make_async_copy(rows0, acc.at[didx.at[0]], c0).wait()
        pltpu.make_async_copy(rows1, acc.at[didx.at[1]], c1).wait()
        return carry

    lax.fori_loop(0, STEPS // GS, group, 0, unroll=False)
    plsc.subcore_barrier()
    pltpu.sync_copy(acc.at[pl.ds(t * RPT, RPT)], out_hbm.at[c, pl.ds(t * RPT, RPT)])


def _dinv(degp_ref):
    deg = degp_ref[0, :, 0] + degp_ref[1, :, 0] + 1.0
    return lax.rsqrt(deg)


def _mm(a, b):
    return lax.dot_general(a, b, (((1,), (1,)), ((), ())),
                           preferred_element_type=jnp.float32)


def _tc1_body(degp_ref, x_ref, w_ref, o_ref):
    dinv = _dinv(degp_ref)
    o_ref[...] = _mm(x_ref[...], w_ref[...]) * dinv[:, None]


def _tc2_body(degp_ref, p0_ref, p1_ref, u_ref, b_ref, w_ref, o_ref):
    dinv = _dinv(degp_ref)
    agg = p0_ref[...] + p1_ref[...] + u_ref[...]
    h = jnp.maximum(agg * dinv[:, None] + b_ref[...], 0.0)
    o_ref[...] = _mm(h, w_ref[...]) * dinv[:, None]


def _tc3_body(degp_ref, p0_ref, p1_ref, u_ref, b_ref, o_ref):
    dinv = _dinv(degp_ref)
    agg = p0_ref[...] + p1_ref[...] + u_ref[...]
    o_ref[...] = agg * dinv[:, None] + b_ref[...]


_deg_spec = pl.BlockSpec((2, RB, D), lambda i: (0, i, 0))
_row_spec = pl.BlockSpec((RB, D), lambda i: (i, 0))
_full_spec = pl.BlockSpec((D, D), lambda i: (0, 0))
_bias_spec = pl.BlockSpec((1, D), lambda i: (0, 0))
_grid = (N // RB,)
_out_rows = jax.ShapeDtypeStruct((N, D), jnp.float32)
_parallel = pltpu.CompilerParams(
    dimension_semantics=("arbitrary",))

_tc1 = pl.pallas_call(
    _tc1_body, grid=_grid,
    in_specs=[_deg_spec, _row_spec, _full_spec],
    out_specs=_row_spec, out_shape=_out_rows, compiler_params=_parallel)

_tc2 = pl.pallas_call(
    _tc2_body, grid=_grid,
    in_specs=[_deg_spec, _row_spec, _row_spec, _row_spec, _bias_spec, _full_spec],
    out_specs=_row_spec, out_shape=_out_rows, compiler_params=_parallel)

_tc3 = pl.pallas_call(
    _tc3_body, grid=_grid,
    in_specs=[_deg_spec, _row_spec, _row_spec, _row_spec, _bias_spec],
    out_specs=_row_spec, out_shape=_out_rows, compiler_params=_parallel)


def kernel(x, edge_index, num_nodes, W1, b1, W2, b2):
    ei = edge_index.astype(jnp.int32)
    r, c = ei[0], ei[1]
    ar = jnp.arange(PAD, dtype=jnp.int32)
    pad_dst = N + (ar % NDUMP)
    pad_src = ar % N
    dst = jnp.concatenate([r, c, pad_dst]).reshape(NW, STEPS, CHUNK)
    src = jnp.concatenate([c, r, pad_src]).reshape(NW, STEPS, CHUNK)

    onesD = jnp.ones((CHUNK, D), jnp.float32)
    zerosD = jnp.zeros((RPT, D), jnp.float32)
    b1r = b1.reshape(1, D)
    b2r = b2.reshape(1, D)

    degp = _deg_sc(dst, onesD, zerosD)
    u1 = _tc1(degp, x, W1)
    p1 = _agg_sc(u1, src, dst, zerosD)
    u2 = _tc2(degp, p1[0, :N], p1[1, :N], u1, b1r, W2)
    p2 = _agg_sc(u2, src, dst, zerosD)
    out = _tc3(degp, p2[0, :N], p2[1, :N], u2, b2r)
    return out


# speedup vs baseline: 1.2163x; 1.2163x over previous
import functools

import jax
import jax.numpy as jnp
from jax import lax
from jax.experimental import pallas as pl
from jax.experimental.pallas import tpu as pltpu
from jax.experimental.pallas import tpu_sc as plsc

N = 10000
D = 128
E0 = 320000
E = 2 * E0
NC = 2
NS = 16
NW = NC * NS
CHUNK = 128
STEPS = 160
GS = 40
EP = NW * STEPS * CHUNK
PAD = EP - E
NDUMP = 240
NP = N + NDUMP
RPT = NP // NS
RB = 400

_mesh = plsc.VectorSubcoreMesh(core_axis_name="c", subcore_axis_name="s")


def _wid():
    return lax.axis_index("c") * NS + lax.axis_index("s")


@functools.partial(
    pl.kernel,
    out_type=jax.ShapeDtypeStruct((NC, NP, D), jnp.float32),
    mesh=_mesh,
    scratch_types=[
        pltpu.VMEM((STEPS, CHUNK), jnp.int32),
        pltpu.VMEM((CHUNK, D), jnp.float32),
        pltpu.VMEM_SHARED((NP, D), jnp.float32),
    ],
    name="gcn_deg_sc",
)
def _deg_sc(dst_hbm, ones_hbm, zeros_hbm, out_hbm, didx, ones_v, acc):
    c = lax.axis_index("c")
    t = lax.axis_index("s")
    w = _wid()
    pltpu.sync_copy(dst_hbm.at[w], didx)
    pltpu.sync_copy(ones_hbm, ones_v)
    pltpu.sync_copy(zeros_hbm, acc.at[pl.ds(t * RPT, RPT)])
    plsc.subcore_barrier()

    def body(j, carry):
        pltpu.sync_copy(ones_v, acc.at[didx.at[j]], add=True)
        return carry

    lax.fori_loop(0, STEPS, body, 0, unroll=False)
    plsc.subcore_barrier()
    pltpu.sync_copy(acc.at[pl.ds(t * RPT, RPT)], out_hbm.at[c, pl.ds(t * RPT, RPT)])


@functools.partial(
    pl.kernel,
    out_type=jax.ShapeDtypeStruct((NC, NP, D), jnp.float32),
    mesh=_mesh,
    scratch_types=[
        pltpu.VMEM((GS, CHUNK), jnp.int32),
        pltpu.VMEM((GS, CHUNK), jnp.int32),
        pltpu.VMEM((CHUNK, D), jnp.float32),
        pltpu.VMEM((CHUNK, D), jnp.float32),
        pltpu.VMEM_SHARED((NP, D), jnp.float32),
        pltpu.SemaphoreType.DMA,
        pltpu.SemaphoreType.DMA,
        pltpu.SemaphoreType.DMA,
        pltpu.SemaphoreType.DMA,
    ],
    name="gcn_agg_sc",
)
def _agg_sc(u_hbm, src_hbm, dst_hbm, zeros_hbm, out_hbm,
            sidx, didx, rows0, rows1, acc, g0, g1, c0, c1):
    c = lax.axis_index("c")
    t = lax.axis_index("s")
    w = _wid()
    pltpu.sync_copy(zeros_hbm, acc.at[pl.ds(t * RPT, RPT)])
    plsc.subcore_barrier()

    def group(g, carry):
        pltpu.sync_copy(src_hbm.at[w, pl.ds(g * GS, GS)], sidx)
        pltpu.sync_copy(dst_hbm.at[w, pl.ds(g * GS, GS)], didx)
        pltpu.async_copy(u_hbm.at[sidx.at[0]], rows0, g0)
        pltpu.async_copy(u_hbm.at[sidx.at[1]], rows1, g1)

        def body(i, carry):
            j0 = 2 * i
            pltpu.make_async_copy(u_hbm.at[sidx.at[j0]], rows0, g0).wait()
            pltpu.async_copy(u_hbm.at[sidx.at[j0 + 1]], rows1, g1)
            pltpu.sync_copy(rows0, acc.at[didx.at[j0]], add=True)
            pltpu.make_async_copy(u_hbm.at[sidx.at[j0 + 1]], rows1, g1).wait()

            @pl.when(i < GS // 2 - 1)
            def _():
                pltpu.async_copy(u_hbm.at[sidx.at[j0 + 2]], rows0, g0)

            pltpu.sync_copy(rows1, acc.at[didx.at[j0 + 1]], add=True)
            return carry

        lax.fori_loop(0, GS // 2, body, 0, unroll=False)
        return carry

    lax.fori_loop(0, STEPS // GS, group, 0, unroll=False)
    plsc.subcore_barrier()
    pltpu.sync_copy(acc.at[pl.ds(t * RPT, RPT)], out_hbm.at[c, pl.ds(t * RPT, RPT)])


def _dinv(degp_ref):
    deg = degp_ref[0, :, 0] + degp_ref[1, :, 0] + 1.0
    return lax.rsqrt(deg)


def _mm(a, b):
    return lax.dot_general(a, b, (((1,), (1,)), ((), ())),
                           preferred_element_type=jnp.float32)


def _tc1_body(degp_ref, x_ref, w_ref, o_ref, dv_ref):
    dinv = _dinv(degp_ref)
    o_ref[...] = _mm(x_ref[...], w_ref[...]) * dinv[:, None]
    dv_ref[...] = dinv[:, None] * jnp.ones((1, 16), jnp.float32)


def _tc2_body(dv_ref, p0_ref, p1_ref, u_ref, b_ref, w_ref, o_ref):
    dinv = dv_ref[:, 0]
    agg = p0_ref[...] + p1_ref[...] + u_ref[...]
    h = jnp.maximum(agg * dinv[:, None] + b_ref[...], 0.0)
    o_ref[...] = _mm(h, w_ref[...]) * dinv[:, None]


def _tc3_body(dv_ref, p0_ref, p1_ref, u_ref, b_ref, o_ref):
    dinv = dv_ref[:, 0]
    agg = p0_ref[...] + p1_ref[...] + u_ref[...]
    o_ref[...] = agg * dinv[:, None] + b_ref[...]


_deg_spec = pl.BlockSpec((2, RB, D), lambda i: (0, i, 0))
_dinv_spec = pl.BlockSpec((RB, 16), lambda i: (i, 0))
_row_spec = pl.BlockSpec((RB, D), lambda i: (i, 0))
_full_spec = pl.BlockSpec((D, D), lambda i: (0, 0))
_bias_spec = pl.BlockSpec((1, D), lambda i: (0, 0))
_grid = (N // RB,)
_out_rows = jax.ShapeDtypeStruct((N, D), jnp.float32)
_parallel = pltpu.CompilerParams(
    dimension_semantics=("arbitrary",))

_tc1 = pl.pallas_call(
    _tc1_body, grid=_grid,
    in_specs=[_deg_spec, _row_spec, _full_spec],
    out_specs=[_row_spec, _dinv_spec],
    out_shape=[_out_rows, jax.ShapeDtypeStruct((N, 16), jnp.float32)],
    compiler_params=_parallel)

_tc2 = pl.pallas_call(
    _tc2_body, grid=_grid,
    in_specs=[_dinv_spec, _row_spec, _row_spec, _row_spec, _bias_spec, _full_spec],
    out_specs=_row_spec, out_shape=_out_rows, compiler_params=_parallel)

_tc3 = pl.pallas_call(
    _tc3_body, grid=_grid,
    in_specs=[_dinv_spec, _row_spec, _row_spec, _row_spec, _bias_spec],
    out_specs=_row_spec, out_shape=_out_rows, compiler_params=_parallel)


def kernel(x, edge_index, num_nodes, W1, b1, W2, b2):
    ei = edge_index.astype(jnp.int32)
    r, c = ei[0], ei[1]
    ar = jnp.arange(PAD, dtype=jnp.int32)
    pad_dst = N + (ar % NDUMP)
    pad_src = ar % N
    dst = jnp.concatenate([r, c, pad_dst]).reshape(NW, STEPS, CHUNK)
    src = jnp.concatenate([c, r, pad_src]).reshape(NW, STEPS, CHUNK)

    onesD = jnp.ones((CHUNK, D), jnp.float32)
    zerosD = jnp.zeros((RPT, D), jnp.float32)
    b1r = b1.reshape(1, D)
    b2r = b2.reshape(1, D)

    degp = _deg_sc(dst, onesD, zerosD)
    u1, dv = _tc1(degp, x, W1)
    p1 = _agg_sc(u1, src, dst, zerosD)
    u2 = _tc2(dv, p1[0, :N], p1[1, :N], u1, b1r, W2)
    p2 = _agg_sc(u2, src, dst, zerosD)
    out = _tc3(dv, p2[0, :N], p2[1, :N], u2, b2r)
    return out
